# baseline (speedup 1.0000x reference)
import jax
import jax.numpy as jnp
from jax.experimental import pallas as pl

ITEMS = 100000
D = 32
B = 8
PACK = 128 // D
ROWS = ITEMS // PACK
BLKR = 5000


def _blend_body(x_ref, e_ref, g_ref, o_ref):
    x = x_ref[...]
    e = e_ref[...]
    g = g_ref[...]
    o_ref[...] = e + g * (x - e)


def kernel(nodes_output, embedding_table, alpha):
    nodes = nodes_output.reshape(B * ROWS, PACK * D)
    emb = embedding_table.reshape(ROWS, PACK * D)
    gate = jnp.broadcast_to(
        alpha.reshape(ROWS, PACK, 1), (ROWS, PACK, D)
    ).reshape(ROWS, PACK * D)
    nblk = ROWS // BLKR
    out = pl.pallas_call(
        _blend_body,
        grid=(nblk, B),
        in_specs=[
            pl.BlockSpec((BLKR, PACK * D), lambda i, b: (b * (ROWS // BLKR) + i, 0)),
            pl.BlockSpec((BLKR, PACK * D), lambda i, b: (i, 0)),
            pl.BlockSpec((BLKR, PACK * D), lambda i, b: (i, 0)),
        ],
        out_specs=pl.BlockSpec(
            (BLKR, PACK * D), lambda i, b: (b * (ROWS // BLKR) + i, 0)),
        out_shape=jax.ShapeDtypeStruct((B * ROWS, PACK * D), jnp.float32),
    )(nodes, emb, gate)
    return out.reshape(B, ITEMS, D)


# speedup vs baseline: 1.1139x; 1.1139x over previous
import jax
import jax.numpy as jnp
from jax import lax
from jax.experimental import pallas as pl
from jax.experimental.pallas import tpu as pltpu

ITEMS = 100000
D = 32
B = 8
BLK = 2000
NI = ITEMS // BLK


def _body(nodes_hbm, emb_hbm, alpha_hbm, out_hbm,
          xbuf, ebuf, abuf, ybuf, xsem, esem, asem, ysem):

    def in_copies(i, slot):
        copies = []
        for b in range(B):
            copies.append(pltpu.make_async_copy(
                nodes_hbm.at[pl.ds(b * ITEMS + i * BLK, BLK)],
                xbuf.at[slot, b], xsem.at[slot, b]))
        copies.append(pltpu.make_async_copy(
            emb_hbm.at[pl.ds(i * BLK, BLK)], ebuf.at[slot], esem.at[slot]))
        copies.append(pltpu.make_async_copy(
            alpha_hbm.at[pl.ds(i * BLK, BLK)], abuf.at[slot], asem.at[slot]))
        return copies

    def out_copies(i, slot):
        return [pltpu.make_async_copy(
            ybuf.at[slot, b], out_hbm.at[b, pl.ds(i * BLK, BLK)],
            ysem.at[slot, b]) for b in range(B)]

    for c in in_copies(0, 0):
        c.start()

    def step(i, slot, first, last):
        @pl.when(jnp.logical_not(last))
        def _():
            for c in in_copies(i + 1, 1 - slot):
                c.start()

        for c in in_copies(i, slot):
            c.wait()

        @pl.when(jnp.logical_not(first))
        def _():
            for c in out_copies(i, slot):
                c.wait()

        e = ebuf[slot]
        a = abuf[slot]
        for b in range(B):
            x = xbuf[slot, b]
            ybuf[slot, b] = e + a * (x - e)
        for c in out_copies(i, slot):
            c.start()

    def pair(p, _):
        i0 = 2 * p
        step(i0, 0, p == 0, p < 0)
        step(i0 + 1, 1, p == 0, i0 + 1 == NI - 1)
        return 0

    lax.fori_loop(0, NI // 2, pair, 0)
    for slot in (0, 1):
        for c in out_copies(0, slot):
            c.wait()


def kernel(nodes_output, embedding_table, alpha):
    return pl.pallas_call(
        _body,
        in_specs=[
            pl.BlockSpec(memory_space=pl.ANY),
            pl.BlockSpec(memory_space=pl.ANY),
            pl.BlockSpec(memory_space=pl.ANY),
        ],
        out_specs=pl.BlockSpec(memory_space=pl.ANY),
        out_shape=jax.ShapeDtypeStruct((B, ITEMS, D), jnp.float32),
        scratch_shapes=[
            pltpu.VMEM((2, B, BLK, D), jnp.float32),
            pltpu.VMEM((2, BLK, D), jnp.float32),
            pltpu.VMEM((2, BLK, 1), jnp.float32),
            pltpu.VMEM((2, B, BLK, D), jnp.float32),
            pltpu.SemaphoreType.DMA((2, B)),
            pltpu.SemaphoreType.DMA((2,)),
            pltpu.SemaphoreType.DMA((2,)),
            pltpu.SemaphoreType.DMA((2, B)),
        ],
    )(nodes_output, embedding_table, alpha)


# speedup vs baseline: 1.3707x; 1.2305x over previous
import jax
import jax.numpy as jnp
from jax.experimental import pallas as pl

ITEMS = 100000
D = 32
B = 8
BLK = 2000


def _blend_body(x_ref, e_ref, a_ref, o_ref):
    x = x_ref[...]
    e = e_ref[...]
    a = a_ref[...]
    o_ref[...] = e[None, :, :] + a[None, :, :] * (x - e[None, :, :])


def kernel(nodes_output, embedding_table, alpha):
    nodes = nodes_output.reshape(B, ITEMS, D)
    grid = (ITEMS // BLK,)
    return pl.pallas_call(
        _blend_body,
        grid=grid,
        in_specs=[
            pl.BlockSpec((B, BLK, D), lambda i: (0, i, 0)),
            pl.BlockSpec((BLK, D), lambda i: (i, 0)),
            pl.BlockSpec((BLK, 1), lambda i: (i, 0)),
        ],
        out_specs=pl.BlockSpec((B, BLK, D), lambda i: (0, i, 0)),
        out_shape=jax.ShapeDtypeStruct((B, ITEMS, D), jnp.float32),
    )(nodes, embedding_table, alpha)
